# baseline (speedup 1.0000x reference)
import functools

import jax
import jax.numpy as jnp
from jax import lax
from jax.experimental import pallas as pl
from jax.experimental.pallas import tpu as pltpu
from jax.experimental.pallas import tpu_sc as plsc

BATCH = 16384
EMBED_DIM = 32

_info = plsc.get_sparse_core_info()
_NC, _NS = _info.num_cores, _info.num_subcores
_NW = _NC * _NS
_BPW = BATCH // _NW
_CHUNK = 128
_NCHUNK = _BPW // _CHUNK


def _sc_gather_body(uidx_hbm, iidx_hbm, gu_tab, gi_tab, mu_tab, mi_tab,
                    gu_out, gi_out, mu_out, mi_out,
                    uidx_v, iidx_v, gu_v, gi_v, mu_v, mi_v, sem):
    wid = lax.axis_index("s") * _NC + lax.axis_index("c")
    base = wid * _BPW
    for j in range(_NCHUNK):
        pltpu.sync_copy(uidx_hbm.at[pl.ds(base + j * _CHUNK, _CHUNK)],
                        uidx_v.at[j])
        pltpu.sync_copy(iidx_hbm.at[pl.ds(base + j * _CHUNK, _CHUNK)],
                        iidx_v.at[j])
    copies = []
    for tab, idx_v, rows_v in ((gu_tab, uidx_v, gu_v),
                               (gi_tab, iidx_v, gi_v),
                               (mu_tab, uidx_v, mu_v),
                               (mi_tab, iidx_v, mi_v)):
        for j in range(_NCHUNK):
            copies.append(pltpu.async_copy(
                tab.at[idx_v.at[j]],
                rows_v.at[pl.ds(j * _CHUNK, _CHUNK)], sem))
    for c in copies:
        c.wait()
    for rows_v, out in ((gu_v, gu_out), (gi_v, gi_out),
                        (mu_v, mu_out), (mi_v, mi_out)):
        pltpu.sync_copy(rows_v, out.at[pl.ds(base, _BPW)])


def _sc_gather(uidx, iidx, gu_tab, gi_tab, mu_tab, mi_tab):
    row_t = jax.ShapeDtypeStruct((BATCH, EMBED_DIM), jnp.float32)
    k = pl.kernel(
        _sc_gather_body,
        out_type=(row_t, row_t, row_t, row_t),
        mesh=plsc.VectorSubcoreMesh(core_axis_name="c", subcore_axis_name="s"),
        scratch_types=[
            pltpu.VMEM((_NCHUNK, _CHUNK), jnp.int32),
            pltpu.VMEM((_NCHUNK, _CHUNK), jnp.int32),
            pltpu.VMEM((_BPW, EMBED_DIM), jnp.float32),
            pltpu.VMEM((_BPW, EMBED_DIM), jnp.float32),
            pltpu.VMEM((_BPW, EMBED_DIM), jnp.float32),
            pltpu.VMEM((_BPW, EMBED_DIM), jnp.float32),
            pltpu.SemaphoreType.DMA,
        ],
        compiler_params=pltpu.CompilerParams(use_tc_tiling_on_sc=False),
    )
    return k(uidx, iidx, gu_tab, gi_tab, mu_tab, mi_tab)


def _mlp_body(gu, gi, mu, mi, w1t, b1r, w2t, b2r, w3t, b3r, wog, woh, bor,
              out):
    x = jnp.concatenate([mu[...], mi[...]], axis=1)
    h = jnp.maximum(jnp.dot(x, w1t[...]) + b1r[...], 0.0)
    h = jnp.maximum(jnp.dot(h, w2t[...]) + b2r[...], 0.0)
    h = jnp.maximum(jnp.dot(h, w3t[...]) + b3r[...], 0.0)
    g = gu[...] * gi[...]
    logits = (jnp.sum(g * wog[...], axis=1, keepdims=True)
              + jnp.sum(h * woh[...], axis=1, keepdims=True)
              + bor[...])
    out[...] = jax.nn.sigmoid(logits)[:, 0]


def kernel(user_indices, item_indices, gmf_user_table, gmf_item_table,
           mlp_user_table, mlp_item_table, W1, b1, W2, b2, W3, b3, Wo, bo):
    uidx = user_indices.astype(jnp.int32)
    iidx = item_indices.astype(jnp.int32)
    gu, gi, mu, mi = _sc_gather(uidx, iidx, gmf_user_table, gmf_item_table,
                                mlp_user_table, mlp_item_table)

    blk = 2048
    grid = BATCH // blk
    row_spec = pl.BlockSpec((blk, EMBED_DIM), lambda i: (i, 0))
    full = lambda s: pl.BlockSpec(s, lambda i: (0,) * len(s))
    out = pl.pallas_call(
        _mlp_body,
        grid=(grid,),
        in_specs=[row_spec, row_spec, row_spec, row_spec,
                  full((64, 32)), full((1, 32)),
                  full((32, 16)), full((1, 16)),
                  full((16, 8)), full((1, 8)),
                  full((1, 32)), full((1, 8)), full((1, 1))],
        out_specs=pl.BlockSpec((blk,), lambda i: (i,)),
        out_shape=jax.ShapeDtypeStruct((BATCH,), jnp.float32),
        compiler_params=pltpu.CompilerParams(
            dimension_semantics=("parallel",)),
    )(gu, gi, mu, mi,
      W1.T, b1.reshape(1, 32),
      W2.T, b2.reshape(1, 16),
      W3.T, b3.reshape(1, 8),
      Wo[:, :32], Wo[:, 32:], bo.reshape(1, 1))
    return out


# speedup vs baseline: 1.4197x; 1.4197x over previous
import jax
import jax.numpy as jnp
from jax import lax
from jax.experimental import pallas as pl
from jax.experimental.pallas import tpu as pltpu
from jax.experimental.pallas import tpu_sc as plsc

BATCH = 16384
EMBED_DIM = 32

_info = plsc.get_sparse_core_info()
_NC, _NS = _info.num_cores, _info.num_subcores
_NW = _NC * _NS
_BPW = BATCH // _NW
_CHUNK = 128
_NCHUNK = _BPW // _CHUNK
_L = 16


def _sc_gather_body(uidx_hbm, iidx_hbm, gu_tab, gi_tab, mu_tab, mi_tab,
                    gu_out, gi_out, mu_out, mi_out,
                    uidx_v, iidx_v, gu_v, gi_v, mu_v, mi_v, sem):
    wid = lax.axis_index("s") * _NC + lax.axis_index("c")
    base = wid * _BPW
    pltpu.sync_copy(uidx_hbm.at[pl.ds(base, _BPW)], uidx_v)
    pltpu.sync_copy(iidx_hbm.at[pl.ds(base, _BPW)], iidx_v)
    for c in range(_NCHUNK):
        def group(g, _):
            off = c * _CHUNK + g * _L
            uv = uidx_v[pl.ds(off, _L)]
            iv = iidx_v[pl.ds(off, _L)]
            for l in range(_L):
                dst = pl.ds(g * _L + l, 1)
                pltpu.async_copy(gu_tab.at[pl.ds(uv[l], 1)], gu_v.at[dst], sem)
                pltpu.async_copy(mu_tab.at[pl.ds(uv[l], 1)], mu_v.at[dst], sem)
                pltpu.async_copy(gi_tab.at[pl.ds(iv[l], 1)], gi_v.at[dst], sem)
                pltpu.async_copy(mi_tab.at[pl.ds(iv[l], 1)], mi_v.at[dst], sem)
            return ()
        lax.fori_loop(0, _CHUNK // _L, group, ())
        pltpu.make_async_copy(gu_tab.at[pl.ds(0, _CHUNK)], gu_v, sem).wait()
        pltpu.make_async_copy(mu_tab.at[pl.ds(0, _CHUNK)], mu_v, sem).wait()
        pltpu.make_async_copy(gi_tab.at[pl.ds(0, _CHUNK)], gi_v, sem).wait()
        pltpu.make_async_copy(mi_tab.at[pl.ds(0, _CHUNK)], mi_v, sem).wait()
        dst = pl.ds(base + c * _CHUNK, _CHUNK)
        pltpu.sync_copy(gu_v, gu_out.at[dst])
        pltpu.sync_copy(gi_v, gi_out.at[dst])
        pltpu.sync_copy(mu_v, mu_out.at[dst])
        pltpu.sync_copy(mi_v, mi_out.at[dst])


def _sc_gather(uidx, iidx, gu_tab, gi_tab, mu_tab, mi_tab):
    row_t = jax.ShapeDtypeStruct((BATCH, EMBED_DIM), jnp.float32)
    k = pl.kernel(
        _sc_gather_body,
        out_type=(row_t, row_t, row_t, row_t),
        mesh=plsc.VectorSubcoreMesh(core_axis_name="c", subcore_axis_name="s"),
        scratch_types=[
            pltpu.VMEM((_BPW,), jnp.int32),
            pltpu.VMEM((_BPW,), jnp.int32),
            pltpu.VMEM((_CHUNK, EMBED_DIM), jnp.float32),
            pltpu.VMEM((_CHUNK, EMBED_DIM), jnp.float32),
            pltpu.VMEM((_CHUNK, EMBED_DIM), jnp.float32),
            pltpu.VMEM((_CHUNK, EMBED_DIM), jnp.float32),
            pltpu.SemaphoreType.DMA,
        ],
    )
    return k(uidx, iidx, gu_tab, gi_tab, mu_tab, mi_tab)


def _mlp_body(gu, gi, mu, mi, w1t, b1r, w2t, b2r, w3t, b3r, wog, woh, bor,
              out):
    x = jnp.concatenate([mu[...], mi[...]], axis=1)
    h = jnp.maximum(jnp.dot(x, w1t[...]) + b1r[...], 0.0)
    h = jnp.maximum(jnp.dot(h, w2t[...]) + b2r[...], 0.0)
    h = jnp.maximum(jnp.dot(h, w3t[...]) + b3r[...], 0.0)
    g = gu[...] * gi[...]
    logits = (jnp.sum(g * wog[...], axis=1, keepdims=True)
              + jnp.sum(h * woh[...], axis=1, keepdims=True)
              + bor[...])
    out[...] = jax.nn.sigmoid(logits)[:, 0]


def kernel(user_indices, item_indices, gmf_user_table, gmf_item_table,
           mlp_user_table, mlp_item_table, W1, b1, W2, b2, W3, b3, Wo, bo):
    uidx = user_indices.astype(jnp.int32)
    iidx = item_indices.astype(jnp.int32)
    gu, gi, mu, mi = _sc_gather(uidx, iidx, gmf_user_table, gmf_item_table,
                                mlp_user_table, mlp_item_table)

    blk = 2048
    grid = BATCH // blk
    row_spec = pl.BlockSpec((blk, EMBED_DIM), lambda i: (i, 0))
    full = lambda s: pl.BlockSpec(s, lambda i: (0,) * len(s))
    out = pl.pallas_call(
        _mlp_body,
        grid=(grid,),
        in_specs=[row_spec, row_spec, row_spec, row_spec,
                  full((64, 32)), full((1, 32)),
                  full((32, 16)), full((1, 16)),
                  full((16, 8)), full((1, 8)),
                  full((1, 32)), full((1, 8)), full((1, 1))],
        out_specs=pl.BlockSpec((blk,), lambda i: (i,)),
        out_shape=jax.ShapeDtypeStruct((BATCH,), jnp.float32),
        compiler_params=pltpu.CompilerParams(
            dimension_semantics=("parallel",)),
    )(gu, gi, mu, mi,
      W1.T, b1.reshape(1, 32),
      W2.T, b2.reshape(1, 16),
      W3.T, b3.reshape(1, 8),
      Wo[:, :32], Wo[:, 32:], bo.reshape(1, 1))
    return out


# speedup vs baseline: 3.2532x; 2.2914x over previous
import jax
import jax.numpy as jnp
from jax import lax
from jax.experimental import pallas as pl
from jax.experimental.pallas import tpu as pltpu
from jax.experimental.pallas import tpu_sc as plsc

BATCH = 16384
EMBED_DIM = 32
NROWS = 1000000

_info = plsc.get_sparse_core_info()
_NC, _NS = _info.num_cores, _info.num_subcores
_NW = _NC * _NS
_BPW = BATCH // _NW
_CI = 16
_NCH = _BPW // _CI


def _sc_gather_body(uidx_hbm, iidx_hbm, guT, giT, muT, miT,
                    gu_out, gi_out, mu_out, mi_out,
                    uidx_v, iidx_v, stage, ring, sem, sem2):
    wid = lax.axis_index("s") * _NC + lax.axis_index("c")
    base = wid * _BPW
    pltpu.sync_copy(uidx_hbm.at[pl.ds(base, _BPW)], uidx_v)
    pltpu.sync_copy(iidx_hbm.at[pl.ds(base, _BPW)], iidx_v)
    f0 = lax.iota(jnp.int32, 16)
    f1 = f0 + 16
    for tabT, idx_v, out in ((guT, uidx_v, gu_out), (muT, uidx_v, mu_out),
                             (giT, iidx_v, gi_out), (miT, iidx_v, mi_out)):
        def chunk(ch, _):
            iv = idx_v[pl.ds(ch * _CI, _CI)]
            for l in range(_CI):
                u = iv[l]
                t128 = pl.multiple_of((u >> 7) << 7, 128)
                pltpu.async_copy(tabT.at[:, pl.ds(t128, 128)],
                                 stage.at[:, pl.ds(l * 128, 128)], sem)
            for l in range(_CI):
                pltpu.make_async_copy(tabT.at[:, pl.ds(0, 128)],
                                      stage.at[:, pl.ds(l * 128, 128)],
                                      sem).wait()
            for l in range(_CI):
                col = (iv[l] & 127) + l * 128
                colv = jnp.zeros((16,), jnp.int32) + col
                lo = plsc.load_gather(stage, [f0, colv])
                hi = plsc.load_gather(stage, [f1, colv])
                row = ring.at[l]
                row[pl.ds(0, 16)] = lo
                row[pl.ds(16, 16)] = hi
                pltpu.async_copy(ring.at[pl.ds(l, 1)],
                                 out.at[pl.ds(base + ch * _CI + l, 1)], sem2)
            for l in range(_CI):
                pltpu.make_async_copy(out.at[pl.ds(0, 1)],
                                      ring.at[pl.ds(l, 1)], sem2).wait()
            return ()
        lax.fori_loop(0, _NCH, chunk, ())


def _sc_gather(uidx, iidx, guT, giT, muT, miT):
    row_t = jax.ShapeDtypeStruct((BATCH, EMBED_DIM), jnp.float32)
    k = pl.kernel(
        _sc_gather_body,
        out_type=(row_t, row_t, row_t, row_t),
        mesh=plsc.VectorSubcoreMesh(core_axis_name="c", subcore_axis_name="s"),
        scratch_types=[
            pltpu.VMEM((_BPW,), jnp.int32),
            pltpu.VMEM((_BPW,), jnp.int32),
            pltpu.VMEM((EMBED_DIM, _CI * 128), jnp.float32),
            pltpu.VMEM((_CI, EMBED_DIM), jnp.float32),
            pltpu.SemaphoreType.DMA,
            pltpu.SemaphoreType.DMA,
        ],
        compiler_params=pltpu.CompilerParams(disable_bounds_checks=True,
                                             needs_layout_passes=False),
    )
    return k(uidx, iidx, guT, giT, muT, miT)


def _mlp_body(gu, gi, mu, mi, w1t, b1r, w2t, b2r, w3t, b3r, wog, woh, bor,
              out):
    x = jnp.concatenate([mu[...], mi[...]], axis=1)
    h = jnp.maximum(jnp.dot(x, w1t[...]) + b1r[...], 0.0)
    h = jnp.maximum(jnp.dot(h, w2t[...]) + b2r[...], 0.0)
    h = jnp.maximum(jnp.dot(h, w3t[...]) + b3r[...], 0.0)
    g = gu[...] * gi[...]
    logits = (jnp.sum(g * wog[...], axis=1, keepdims=True)
              + jnp.sum(h * woh[...], axis=1, keepdims=True)
              + bor[...])
    out[...] = jax.nn.sigmoid(logits)[:, 0]


def kernel(user_indices, item_indices, gmf_user_table, gmf_item_table,
           mlp_user_table, mlp_item_table, W1, b1, W2, b2, W3, b3, Wo, bo):
    uidx = user_indices.astype(jnp.int32)
    iidx = item_indices.astype(jnp.int32)
    gu, gi, mu, mi = _sc_gather(uidx, iidx,
                                gmf_user_table.T, gmf_item_table.T,
                                mlp_user_table.T, mlp_item_table.T)

    blk = 2048
    grid = BATCH // blk
    row_spec = pl.BlockSpec((blk, EMBED_DIM), lambda i: (i, 0))
    full = lambda s: pl.BlockSpec(s, lambda i: (0,) * len(s))
    out = pl.pallas_call(
        _mlp_body,
        grid=(grid,),
        in_specs=[row_spec, row_spec, row_spec, row_spec,
                  full((64, 32)), full((1, 32)),
                  full((32, 16)), full((1, 16)),
                  full((16, 8)), full((1, 8)),
                  full((1, 32)), full((1, 8)), full((1, 1))],
        out_specs=pl.BlockSpec((blk,), lambda i: (i,)),
        out_shape=jax.ShapeDtypeStruct((BATCH,), jnp.float32),
        compiler_params=pltpu.CompilerParams(
            dimension_semantics=("parallel",)),
    )(gu, gi, mu, mi,
      W1.T, b1.reshape(1, 32),
      W2.T, b2.reshape(1, 16),
      W3.T, b3.reshape(1, 8),
      Wo[:, :32], Wo[:, 32:], bo.reshape(1, 1))
    return out


# speedup vs baseline: 3.3949x; 1.0435x over previous
import jax
import jax.numpy as jnp
from jax import lax
from jax.experimental import pallas as pl
from jax.experimental.pallas import tpu as pltpu
from jax.experimental.pallas import tpu_sc as plsc

BATCH = 16384
EMBED_DIM = 32
NROWS = 1000000

_info = plsc.get_sparse_core_info()
_NC, _NS = _info.num_cores, _info.num_subcores
_NW = _NC * _NS
_BPW = BATCH // _NW
_CI = 8
_NCH = _BPW // _CI


def _sc_gather_body(uidx_hbm, iidx_hbm, guT, giT, muT, miT,
                    gu_out, gi_out, mu_out, mi_out,
                    uidx_v, iidx_v, stage, ring, sem_a, sem_b, sem2):
    wid = lax.axis_index("s") * _NC + lax.axis_index("c")
    base = wid * _BPW
    pltpu.sync_copy(uidx_hbm.at[pl.ds(base, _BPW)], uidx_v.at[pl.ds(0, _BPW)])
    pltpu.sync_copy(iidx_hbm.at[pl.ds(base, _BPW)], iidx_v.at[pl.ds(0, _BPW)])
    f0 = lax.iota(jnp.int32, 16)
    f1 = f0 + 16
    sems = (sem_a, sem_b)
    for tabT, idx_v, out in ((guT, uidx_v, gu_out), (muT, uidx_v, mu_out),
                             (giT, iidx_v, gi_out), (miT, iidx_v, mi_out)):
        def fire_dyn(ch):
            @pl.when(ch % 2 == 0)
            def _():
                _fire_par(ch, 0)
            @pl.when(ch % 2 == 1)
            def _():
                _fire_par(ch, 1)

        def _fire_par(ch, b):
            iv = idx_v[pl.ds(ch * _CI, 16)]
            for l in range(_CI):
                u = iv[l]
                t128 = pl.multiple_of((u >> 7) << 7, 128)
                pltpu.async_copy(tabT.at[:, pl.ds(t128, 128)],
                                 stage.at[b, :, pl.ds(l * 128, 128)],
                                 sems[b])

        _fire_par(0, 0)

        def chunk(ch, _):
            @pl.when(ch + 1 < _NCH)
            def _():
                fire_dyn(ch + 1)
            @pl.when(ch > 0)
            def _():
                for l in range(_CI):
                    pltpu.make_async_copy(out.at[pl.ds(0, 1)],
                                          ring.at[0, pl.ds(l, 1)],
                                          sem2).wait()
            iv = idx_v[pl.ds(ch * _CI, 16)]
            for b in (0, 1):
                @pl.when(ch % 2 == b)
                def _():
                    for l in range(_CI):
                        pltpu.make_async_copy(
                            tabT.at[:, pl.ds(0, 128)],
                            stage.at[b, :, pl.ds(l * 128, 128)],
                            sems[b]).wait()
                    for l in range(_CI):
                        col = (iv[l] & 127) + l * 128
                        colv = jnp.zeros((16,), jnp.int32) + col
                        lo = plsc.load_gather(stage.at[b], [f0, colv])
                        hi = plsc.load_gather(stage.at[b], [f1, colv])
                        row = ring.at[0, l]
                        row[pl.ds(0, 16)] = lo
                        row[pl.ds(16, 16)] = hi
                        pltpu.async_copy(
                            ring.at[0, pl.ds(l, 1)],
                            out.at[pl.ds(base + ch * _CI + l, 1)], sem2)
            return ()
        lax.fori_loop(0, _NCH, chunk, ())
        for l in range(_CI):
            pltpu.make_async_copy(out.at[pl.ds(0, 1)],
                                  ring.at[0, pl.ds(l, 1)], sem2).wait()


def _sc_gather(uidx, iidx, guT, giT, muT, miT):
    row_t = jax.ShapeDtypeStruct((BATCH, EMBED_DIM), jnp.float32)
    k = pl.kernel(
        _sc_gather_body,
        out_type=(row_t, row_t, row_t, row_t),
        mesh=plsc.VectorSubcoreMesh(core_axis_name="c", subcore_axis_name="s"),
        scratch_types=[
            pltpu.VMEM((_BPW + 16,), jnp.int32),
            pltpu.VMEM((_BPW + 16,), jnp.int32),
            pltpu.VMEM((2, EMBED_DIM, _CI * 128), jnp.float32),
            pltpu.VMEM((1, _CI, EMBED_DIM), jnp.float32),
            pltpu.SemaphoreType.DMA,
            pltpu.SemaphoreType.DMA,
            pltpu.SemaphoreType.DMA,
        ],
        compiler_params=pltpu.CompilerParams(disable_bounds_checks=True,
                                             needs_layout_passes=False),
    )
    return k(uidx, iidx, guT, giT, muT, miT)


def _mlp_body(gu, gi, mu, mi, w1t, b1r, w2t, b2r, w3t, b3r, wog, woh, bor,
              out):
    x = jnp.concatenate([mu[...], mi[...]], axis=1)
    h = jnp.maximum(jnp.dot(x, w1t[...]) + b1r[...], 0.0)
    h = jnp.maximum(jnp.dot(h, w2t[...]) + b2r[...], 0.0)
    h = jnp.maximum(jnp.dot(h, w3t[...]) + b3r[...], 0.0)
    g = gu[...] * gi[...]
    logits = (jnp.sum(g * wog[...], axis=1, keepdims=True)
              + jnp.sum(h * woh[...], axis=1, keepdims=True)
              + bor[...])
    out[...] = jax.nn.sigmoid(logits)[:, 0]


def kernel(user_indices, item_indices, gmf_user_table, gmf_item_table,
           mlp_user_table, mlp_item_table, W1, b1, W2, b2, W3, b3, Wo, bo):
    uidx = user_indices.astype(jnp.int32)
    iidx = item_indices.astype(jnp.int32)
    gu, gi, mu, mi = _sc_gather(uidx, iidx,
                                gmf_user_table.T, gmf_item_table.T,
                                mlp_user_table.T, mlp_item_table.T)

    blk = 2048
    grid = BATCH // blk
    row_spec = pl.BlockSpec((blk, EMBED_DIM), lambda i: (i, 0))
    full = lambda s: pl.BlockSpec(s, lambda i: (0,) * len(s))
    out = pl.pallas_call(
        _mlp_body,
        grid=(grid,),
        in_specs=[row_spec, row_spec, row_spec, row_spec,
                  full((64, 32)), full((1, 32)),
                  full((32, 16)), full((1, 16)),
                  full((16, 8)), full((1, 8)),
                  full((1, 32)), full((1, 8)), full((1, 1))],
        out_specs=pl.BlockSpec((blk,), lambda i: (i,)),
        out_shape=jax.ShapeDtypeStruct((BATCH,), jnp.float32),
        compiler_params=pltpu.CompilerParams(
            dimension_semantics=("parallel",)),
    )(gu, gi, mu, mi,
      W1.T, b1.reshape(1, 32),
      W2.T, b2.reshape(1, 16),
      W3.T, b3.reshape(1, 8),
      Wo[:, :32], Wo[:, 32:], bo.reshape(1, 1))
    return out


# speedup vs baseline: 3.7239x; 1.0969x over previous
import jax
import jax.numpy as jnp
from jax import lax
from jax.experimental import pallas as pl
from jax.experimental.pallas import tpu as pltpu
from jax.experimental.pallas import tpu_sc as plsc

BATCH = 16384
EMBED_DIM = 32
NROWS = 1000000

_info = plsc.get_sparse_core_info()
_NC, _NS = _info.num_cores, _info.num_subcores
_NW = _NC * _NS
_BPW = BATCH // _NW
_CI = 8
_NCH = _BPW // _CI


def _sc_gather_body(uidx_hbm, iidx_hbm, guT, giT, muT, miT,
                    gu_out, gi_out, mu_out, mi_out,
                    uidx_v, iidx_v, stage, ring, sem_a, sem_b, sem_c, sem2):
    wid = lax.axis_index("s") * _NC + lax.axis_index("c")
    base = wid * _BPW
    pltpu.sync_copy(uidx_hbm.at[pl.ds(base, _BPW)], uidx_v.at[pl.ds(0, _BPW)])
    pltpu.sync_copy(iidx_hbm.at[pl.ds(base, _BPW)], iidx_v.at[pl.ds(0, _BPW)])
    f0 = lax.iota(jnp.int32, 16)
    f1 = f0 + 16
    sems = (sem_a, sem_b, sem_c)
    _NB = 3
    for tabT, idx_v, out in ((guT, uidx_v, gu_out), (muT, uidx_v, mu_out),
                             (giT, iidx_v, gi_out), (miT, iidx_v, mi_out)):
        def _fire_par(ch, b):
            iv = idx_v[pl.ds(ch * _CI, 16)]
            for l in range(_CI):
                u = iv[l]
                t128 = pl.multiple_of((u >> 7) << 7, 128)
                pltpu.async_copy(tabT.at[:, pl.ds(t128, 128)],
                                 stage.at[b, :, pl.ds(l * 128, 128)],
                                 sems[b])

        def fire_dyn(ch):
            for b in range(_NB):
                @pl.when(ch % _NB == b)
                def _():
                    _fire_par(ch, b)

        _fire_par(0, 0)
        _fire_par(1, 1)

        def chunk(ch, _):
            @pl.when(ch + 2 < _NCH)
            def _():
                fire_dyn(ch + 2)
            @pl.when(ch > 0)
            def _():
                for l in range(_CI):
                    pltpu.make_async_copy(out.at[pl.ds(0, 1)],
                                          ring.at[0, pl.ds(l, 1)],
                                          sem2).wait()
            iv = idx_v[pl.ds(ch * _CI, 16)]
            for b in range(_NB):
                @pl.when(ch % _NB == b)
                def _():
                    for l in range(_CI):
                        pltpu.make_async_copy(
                            tabT.at[:, pl.ds(0, 128)],
                            stage.at[b, :, pl.ds(l * 128, 128)],
                            sems[b]).wait()
                    for l in range(_CI):
                        col = (iv[l] & 127) + l * 128
                        colv = jnp.zeros((16,), jnp.int32) + col
                        lo = plsc.load_gather(stage.at[b], [f0, colv])
                        hi = plsc.load_gather(stage.at[b], [f1, colv])
                        row = ring.at[0, l]
                        row[pl.ds(0, 16)] = lo
                        row[pl.ds(16, 16)] = hi
                        pltpu.async_copy(
                            ring.at[0, pl.ds(l, 1)],
                            out.at[pl.ds(base + ch * _CI + l, 1)], sem2)
            return ()
        lax.fori_loop(0, _NCH, chunk, ())
        for l in range(_CI):
            pltpu.make_async_copy(out.at[pl.ds(0, 1)],
                                  ring.at[0, pl.ds(l, 1)], sem2).wait()


def _sc_gather(uidx, iidx, guT, giT, muT, miT):
    row_t = jax.ShapeDtypeStruct((BATCH, EMBED_DIM), jnp.float32)
    k = pl.kernel(
        _sc_gather_body,
        out_type=(row_t, row_t, row_t, row_t),
        mesh=plsc.VectorSubcoreMesh(core_axis_name="c", subcore_axis_name="s"),
        scratch_types=[
            pltpu.VMEM((_BPW + 16,), jnp.int32),
            pltpu.VMEM((_BPW + 16,), jnp.int32),
            pltpu.VMEM((3, EMBED_DIM, _CI * 128), jnp.float32),
            pltpu.VMEM((1, _CI, EMBED_DIM), jnp.float32),
            pltpu.SemaphoreType.DMA,
            pltpu.SemaphoreType.DMA,
            pltpu.SemaphoreType.DMA,
            pltpu.SemaphoreType.DMA,
        ],
        compiler_params=pltpu.CompilerParams(disable_bounds_checks=True,
                                             needs_layout_passes=False),
    )
    return k(uidx, iidx, guT, giT, muT, miT)


def _mlp_body(gu, gi, mu, mi, w1t, b1r, w2t, b2r, w3t, b3r, wog, woh, bor,
              out):
    x = jnp.concatenate([mu[...], mi[...]], axis=1)
    h = jnp.maximum(jnp.dot(x, w1t[...]) + b1r[...], 0.0)
    h = jnp.maximum(jnp.dot(h, w2t[...]) + b2r[...], 0.0)
    h = jnp.maximum(jnp.dot(h, w3t[...]) + b3r[...], 0.0)
    g = gu[...] * gi[...]
    logits = (jnp.sum(g * wog[...], axis=1, keepdims=True)
              + jnp.sum(h * woh[...], axis=1, keepdims=True)
              + bor[...])
    out[...] = jax.nn.sigmoid(logits)[:, 0]


def kernel(user_indices, item_indices, gmf_user_table, gmf_item_table,
           mlp_user_table, mlp_item_table, W1, b1, W2, b2, W3, b3, Wo, bo):
    uidx = user_indices.astype(jnp.int32)
    iidx = item_indices.astype(jnp.int32)
    gu, gi, mu, mi = _sc_gather(uidx, iidx,
                                gmf_user_table.T, gmf_item_table.T,
                                mlp_user_table.T, mlp_item_table.T)

    blk = 2048
    grid = BATCH // blk
    row_spec = pl.BlockSpec((blk, EMBED_DIM), lambda i: (i, 0))
    full = lambda s: pl.BlockSpec(s, lambda i: (0,) * len(s))
    out = pl.pallas_call(
        _mlp_body,
        grid=(grid,),
        in_specs=[row_spec, row_spec, row_spec, row_spec,
                  full((64, 32)), full((1, 32)),
                  full((32, 16)), full((1, 16)),
                  full((16, 8)), full((1, 8)),
                  full((1, 32)), full((1, 8)), full((1, 1))],
        out_specs=pl.BlockSpec((blk,), lambda i: (i,)),
        out_shape=jax.ShapeDtypeStruct((BATCH,), jnp.float32),
        compiler_params=pltpu.CompilerParams(
            dimension_semantics=("parallel",)),
    )(gu, gi, mu, mi,
      W1.T, b1.reshape(1, 32),
      W2.T, b2.reshape(1, 16),
      W3.T, b3.reshape(1, 8),
      Wo[:, :32], Wo[:, 32:], bo.reshape(1, 1))
    return out
